# baseline (speedup 1.0000x reference)
import functools

import jax
import jax.numpy as jnp
from jax import lax
from jax.experimental import pallas as pl
from jax.experimental.pallas import tpu as pltpu
from jax.experimental.pallas import tpu_sc as plsc


_BR = 1000


def _linear_body(x_ref, w_ref, b_ref, o_ref):
    o_ref[...] = lax.dot_general(
        x_ref[...], w_ref[...], (((1,), (1,)), ((), ())),
        preferred_element_type=jnp.float32) + b_ref[...]


def _tc_linear(x, w, b):
    n, d = x.shape
    return pl.pallas_call(
        _linear_body,
        grid=(n // _BR,),
        in_specs=[
            pl.BlockSpec((_BR, d), lambda i: (i, 0)),
            pl.BlockSpec((d, d), lambda i: (0, 0)),
            pl.BlockSpec((1, d), lambda i: (0, 0)),
        ],
        out_specs=pl.BlockSpec((_BR, d), lambda i: (i, 0)),
        out_shape=jax.ShapeDtypeStruct((n, d), jnp.float32),
    )(x, w, b.reshape(1, d))


def _combine_body(nps, p_refs_and_rest):
    p_refs = p_refs_and_rest[:nps]
    h_ref, wrel_ref, brel_ref, wroot_ref, o_ref = p_refs_and_rest[nps:]
    agg = p_refs[0][0]
    for pr in p_refs[1:]:
        agg = agg + pr[0]
    acc = lax.dot_general(agg, wrel_ref[...], (((1,), (1,)), ((), ())),
                          preferred_element_type=jnp.float32)
    acc += lax.dot_general(h_ref[...], wroot_ref[...], (((1,), (1,)), ((), ())),
                           preferred_element_type=jnp.float32)
    o_ref[...] = jnp.maximum(acc + brel_ref[...], 0.0)


def _tc_combine(ps, h, w_rel, b_rel, w_root):
    n, d = h.shape
    return pl.pallas_call(
        lambda *refs: _combine_body(len(ps), refs),
        grid=(n // _BR,),
        in_specs=[pl.BlockSpec((1, _BR, d), lambda i: (0, i, 0))
                  for _ in ps] + [
            pl.BlockSpec((_BR, d), lambda i: (i, 0)),
            pl.BlockSpec((d, d), lambda i: (0, 0)),
            pl.BlockSpec((1, d), lambda i: (0, 0)),
            pl.BlockSpec((d, d), lambda i: (0, 0)),
        ],
        out_specs=pl.BlockSpec((_BR, d), lambda i: (i, 0)),
        out_shape=jax.ShapeDtypeStruct((n, d), jnp.float32),
    )(*ps, h, w_rel, b_rel.reshape(1, d), w_root)


_B = 80
_CB = 25
_ZR = 16
_WB = 80


def _sc_segsum_body(ns, nchunks, n,
                    h_hbm, src_hbm, dst_hbm, out_hbm,
                    src_v, dst_v, rows_v, zb_v, acc_s,
                    sg0, sg1, ss, sc0, sc1):
    c = lax.axis_index("c")
    s = lax.axis_index("s")
    wid = c * ns + s
    sgs = (sg0, sg1)
    scs = (sc0, sc1)

    for i in range(_ZR):
        for k in range(zb_v.shape[1] // 16):
            zb_v[i, pl.ds(k * 16, 16)] = jnp.zeros((16,), jnp.float32)
    nwb = n // _WB
    for k in range((nwb + ns - 1) // ns):
        ci = k * ns + s

        @pl.when(ci < nwb)
        def _():
            r0 = ci * _WB
            for m in range(_WB // _ZR):
                pltpu.sync_copy(zb_v, acc_s.at[pl.ds(r0 + m * _ZR, _ZR)])
    plsc.subcore_barrier()

    def stage(cc, p):
        pltpu.async_copy(src_hbm.at[wid, cc], src_v.at[p], ss)
        pltpu.async_copy(dst_hbm.at[wid, cc], dst_v.at[p], ss)

    def drain_stage(p):
        pltpu.make_async_copy(src_hbm.at[wid, 0], src_v.at[p], ss).wait()
        pltpu.make_async_copy(dst_hbm.at[wid, 0], dst_v.at[p], ss).wait()

    def process(p):
        def fire(j):
            return pltpu.async_copy(h_hbm.at[src_v.at[p, j]],
                                    rows_v.at[j % 2], sgs[j % 2])

        gds = [None] * _CB
        sds = [None] * _CB
        gds[0] = fire(0)
        for j in range(_CB):
            if j >= 1:
                sds[j - 1].wait()
            if j + 1 < _CB:
                gds[j + 1] = fire(j + 1)
            gds[j].wait()
            sds[j] = pltpu.async_copy(rows_v.at[j % 2],
                                      acc_s.at[dst_v.at[p, j]],
                                      scs[j % 2], add=True)
        sds[_CB - 1].wait()

    stage(0, 0)

    def body(ci, carry):
        p = lax.rem(ci, 2)
        drain_stage(p)

        @pl.when(ci + 1 < nchunks)
        def _():
            stage(ci + 1, 1 - p)
        process(p)
        return carry

    lax.fori_loop(0, nchunks, body, 0)
    plsc.subcore_barrier()

    for k in range((nwb + ns - 1) // ns):
        ci = k * ns + s

        @pl.when(ci < nwb)
        def _():
            r0 = ci * _WB
            pltpu.sync_copy(acc_s.at[pl.ds(r0, _WB)],
                            out_hbm.at[c, pl.ds(r0, _WB)])


def _sc_segment_sum(h, src4, dst4):
    n, d = h.shape
    nw, nchunks, cb, b = dst4.shape
    info = plsc.get_sparse_core_info()
    ns = info.num_subcores
    assert nw == ns and cb == _CB and b == _B
    assert n % _WB == 0 and _WB % _ZR == 0
    mesh = plsc.VectorSubcoreMesh(core_axis_name="c", subcore_axis_name="s",
                                  num_cores=1)
    kern = pl.kernel(
        functools.partial(_sc_segsum_body, ns, nchunks, n),
        out_type=jax.ShapeDtypeStruct((1, n, d), jnp.float32),
        mesh=mesh,
        scratch_types=[
            pltpu.VMEM((2, _CB, _B), jnp.int32),
            pltpu.VMEM((2, _CB, _B), jnp.int32),
            pltpu.VMEM((2, _B, d), jnp.float32),
            pltpu.VMEM((_ZR, d), jnp.float32),
            pltpu.VMEM_SHARED((n, d), jnp.float32),
            pltpu.SemaphoreType.DMA,
            pltpu.SemaphoreType.DMA,
            pltpu.SemaphoreType.DMA,
            pltpu.SemaphoreType.DMA,
            pltpu.SemaphoreType.DMA,
        ],
    )
    return kern(h, src4, dst4)


def kernel(x, edge_index, W_in, b_in, W_rel1, b_rel1, W_root1,
           W_rel2, b_rel2, W_root2):
    e = edge_index.shape[1]
    info = plsc.get_sparse_core_info()
    ns = info.num_subcores
    half = e // 2
    cedges = _CB * _B
    assert half % (ns * cedges) == 0
    nchunks = half // (ns * cedges)

    def shape4(a):
        return a.reshape(ns, nchunks, _CB, _B)

    srcA = shape4(edge_index[0, :half])
    srcB = shape4(edge_index[0, half:])
    dstA = shape4(edge_index[1, :half])
    dstB = shape4(edge_index[1, half:])

    h0 = _tc_linear(x, W_in, b_in)
    p1a = _sc_segment_sum(h0, srcA, dstA)
    p1b = _sc_segment_sum(h0, srcB, dstB)
    h1 = _tc_combine([p1a, p1b], h0, W_rel1, b_rel1, W_root1)
    p2a = _sc_segment_sum(h1, srcA, dstA)
    p2b = _sc_segment_sum(h1, srcB, dstB)
    h2 = _tc_combine([p2a, p2b], h1, W_rel2, b_rel2, W_root2)
    return h2


# speedup vs baseline: 1.0561x; 1.0561x over previous
import functools

import jax
import jax.numpy as jnp
from jax import lax
from jax.experimental import pallas as pl
from jax.experimental.pallas import tpu as pltpu
from jax.experimental.pallas import tpu_sc as plsc


_BR = 1000


def _linear_body(x_ref, w_ref, b_ref, o_ref):
    o_ref[...] = lax.dot_general(
        x_ref[...], w_ref[...], (((1,), (1,)), ((), ())),
        preferred_element_type=jnp.float32) + b_ref[...]


def _tc_linear(x, w, b):
    n, d = x.shape
    return pl.pallas_call(
        _linear_body,
        grid=(n // _BR,),
        in_specs=[
            pl.BlockSpec((_BR, d), lambda i: (i, 0)),
            pl.BlockSpec((d, d), lambda i: (0, 0)),
            pl.BlockSpec((1, d), lambda i: (0, 0)),
        ],
        out_specs=pl.BlockSpec((_BR, d), lambda i: (i, 0)),
        out_shape=jax.ShapeDtypeStruct((n, d), jnp.float32),
    )(x, w, b.reshape(1, d))


def _combine_body(nps, p_refs_and_rest):
    p_refs = p_refs_and_rest[:nps]
    h_ref, wrel_ref, brel_ref, wroot_ref, o_ref = p_refs_and_rest[nps:]
    agg = p_refs[0][0]
    for pr in p_refs[1:]:
        agg = agg + pr[0]
    acc = lax.dot_general(agg, wrel_ref[...], (((1,), (1,)), ((), ())),
                          preferred_element_type=jnp.float32)
    acc += lax.dot_general(h_ref[...], wroot_ref[...], (((1,), (1,)), ((), ())),
                           preferred_element_type=jnp.float32)
    o_ref[...] = jnp.maximum(acc + brel_ref[...], 0.0)


def _tc_combine(ps, h, w_rel, b_rel, w_root):
    n, d = h.shape
    return pl.pallas_call(
        lambda *refs: _combine_body(len(ps), refs),
        grid=(n // _BR,),
        in_specs=[pl.BlockSpec((1, _BR, d), lambda i: (0, i, 0))
                  for _ in ps] + [
            pl.BlockSpec((_BR, d), lambda i: (i, 0)),
            pl.BlockSpec((d, d), lambda i: (0, 0)),
            pl.BlockSpec((1, d), lambda i: (0, 0)),
            pl.BlockSpec((d, d), lambda i: (0, 0)),
        ],
        out_specs=pl.BlockSpec((_BR, d), lambda i: (i, 0)),
        out_shape=jax.ShapeDtypeStruct((n, d), jnp.float32),
    )(*ps, h, w_rel, b_rel.reshape(1, d), w_root)


_B = 80
_CB = 25
_ZR = 16
_WB = 80


def _sc_segsum_body(ns, nchunks, n,
                    h_hbm, src_hbm, dst_hbm, out_hbm,
                    src_v, dst_v, rows_v, zb_v, acc_s,
                    sg0, sg1, ss, sc0, sc1):
    c = lax.axis_index("c")
    s = lax.axis_index("s")
    wid = c * ns + s
    sgs = (sg0, sg1)
    scs = (sc0, sc1)

    for i in range(_ZR):
        for k in range(zb_v.shape[1] // 16):
            zb_v[i, pl.ds(k * 16, 16)] = jnp.zeros((16,), jnp.float32)
    nwb = n // _WB
    for k in range((nwb + ns - 1) // ns):
        ci = k * ns + s

        @pl.when(ci < nwb)
        def _():
            r0 = ci * _WB
            for m in range(_WB // _ZR):
                pltpu.sync_copy(zb_v, acc_s.at[pl.ds(r0 + m * _ZR, _ZR)])
    plsc.subcore_barrier()

    def stage(cc, p):
        pltpu.async_copy(src_hbm.at[wid, cc], src_v.at[p], ss)
        pltpu.async_copy(dst_hbm.at[wid, cc], dst_v.at[p], ss)

    def drain_stage(p):
        pltpu.make_async_copy(src_hbm.at[wid, 0], src_v.at[p], ss).wait()
        pltpu.make_async_copy(dst_hbm.at[wid, 0], dst_v.at[p], ss).wait()

    def process(p):
        def fire(j):
            return pltpu.async_copy(h_hbm.at[src_v.at[p, j]],
                                    rows_v.at[j % 2], sgs[j % 2])

        gds = [None] * _CB
        sds = [None] * _CB
        gds[0] = fire(0)
        for j in range(_CB):
            if j >= 1:
                sds[j - 1].wait()
            if j + 1 < _CB:
                gds[j + 1] = fire(j + 1)
            gds[j].wait()
            sds[j] = pltpu.async_copy(rows_v.at[j % 2],
                                      acc_s.at[dst_v.at[p, j]],
                                      scs[j % 2], add=True)
        sds[_CB - 1].wait()

    stage(0, 0)

    def body(ci, carry):
        p = lax.rem(ci, 2)
        drain_stage(p)

        @pl.when(ci + 1 < nchunks)
        def _():
            stage(ci + 1, 1 - p)
        process(p)
        return carry

    lax.fori_loop(0, nchunks, body, 0)
    plsc.subcore_barrier()

    for k in range((nwb + ns - 1) // ns):
        ci = k * ns + s

        @pl.when(ci < nwb)
        def _():
            r0 = ci * _WB
            pltpu.sync_copy(acc_s.at[pl.ds(r0, _WB)],
                            out_hbm.at[c, pl.ds(r0, _WB)])


def _sc_segment_sum(h, src4, dst4):
    n, d = h.shape
    nw, nchunks, cb, b = dst4.shape
    info = plsc.get_sparse_core_info()
    ns = info.num_subcores
    assert nw == ns and cb == _CB and b == _B
    assert n % _WB == 0 and _WB % _ZR == 0
    mesh = plsc.VectorSubcoreMesh(core_axis_name="c", subcore_axis_name="s",
                                  num_cores=1)
    kern = pl.kernel(
        functools.partial(_sc_segsum_body, ns, nchunks, n),
        out_type=jax.ShapeDtypeStruct((1, n, d), jnp.float32),
        mesh=mesh,
        scratch_types=[
            pltpu.VMEM((2, _CB, _B), jnp.int32),
            pltpu.VMEM((2, _CB, _B), jnp.int32),
            pltpu.VMEM((2, _B, d), jnp.float32),
            pltpu.VMEM((_ZR, d), jnp.float32),
            pltpu.VMEM_SHARED((n, d), jnp.float32),
            pltpu.SemaphoreType.DMA,
            pltpu.SemaphoreType.DMA,
            pltpu.SemaphoreType.DMA,
            pltpu.SemaphoreType.DMA,
            pltpu.SemaphoreType.DMA,
        ],
    )
    return kern(h, src4, dst4)


def kernel(x, edge_index, W_in, b_in, W_rel1, b_rel1, W_root1,
           W_rel2, b_rel2, W_root2):
    e = edge_index.shape[1]
    info = plsc.get_sparse_core_info()
    ns = info.num_subcores
    cedges = _CB * _B
    assert e % (ns * cedges) == 0
    nchunks = e // (ns * cedges)

    src4 = edge_index[0].reshape(ns, nchunks, _CB, _B)
    dst4 = edge_index[1].reshape(ns, nchunks, _CB, _B)

    h0 = _tc_linear(x, W_in, b_in)
    p1 = _sc_segment_sum(h0, src4, dst4)
    h1 = _tc_combine([p1], h0, W_rel1, b_rel1, W_root1)
    p2 = _sc_segment_sum(h1, src4, dst4)
    h2 = _tc_combine([p2], h1, W_rel2, b_rel2, W_root2)
    return h2
